# baseline (speedup 1.0000x reference)
import jax
import jax.numpy as jnp
from jax import lax
from jax.experimental import pallas as pl
from jax.experimental.pallas import tpu as pltpu
from jax.experimental.pallas import tpu_sc as plsc

N_NODES = 50000
E_EDGES = 800000
A_ACT = 8

_NC = 2
_NS = 16
_NW = _NC * _NS
_CHUNK = 12512
_PER_W = 2 * _CHUNK
_E_PAD = _NW * _PER_W


def _sc_gather_body(src_hbm, dst_hbm, acts_hbm, as_hbm, ad_hbm,
                    src_v, dst_v, as_v, ad_v, sem):
    c = lax.axis_index("c")
    s = lax.axis_index("s")
    wid = s * _NC + c
    base = pl.multiple_of(wid * _PER_W, 8)
    pltpu.sync_copy(src_hbm.at[pl.ds(base, _PER_W)], src_v)
    pltpu.sync_copy(dst_hbm.at[pl.ds(base, _PER_W)], dst_v)
    pltpu.async_copy(acts_hbm.at[src_v], as_v, sem).wait()
    pltpu.async_copy(acts_hbm.at[dst_v], ad_v, sem).wait()
    pltpu.sync_copy(as_v, as_hbm.at[pl.ds(base, _PER_W)])
    pltpu.sync_copy(ad_v, ad_hbm.at[pl.ds(base, _PER_W)])


_sc_gather_built = None


def _sc_gather(src, dst, acts):
    global _sc_gather_built
    if _sc_gather_built is None:
        _sc_gather_built = pl.kernel(
            _sc_gather_body,
            out_type=[jax.ShapeDtypeStruct((_E_PAD,), jnp.int32),
                      jax.ShapeDtypeStruct((_E_PAD,), jnp.int32)],
            mesh=plsc.VectorSubcoreMesh(core_axis_name="c",
                                        subcore_axis_name="s"),
            scratch_types=[
                pltpu.VMEM((_PER_W,), jnp.int32),
                pltpu.VMEM((_PER_W,), jnp.int32),
                pltpu.VMEM((_PER_W,), jnp.int32),
                pltpu.VMEM((_PER_W,), jnp.int32),
                pltpu.SemaphoreType.DMA,
            ],
        )
    return _sc_gather_built(src, dst, acts)

_EB = 16000
_EG = E_EDGES // _EB
_NB = 5000
_NG = N_NODES // _NB

_CONTRACT_MINOR = (((1,), (1,)), ((), ()))


def _mlp2(x, w1, b1, w3, b3):
    h = lax.dot_general(x, w1, _CONTRACT_MINOR,
                        preferred_element_type=jnp.float32) + b1
    h = jnp.maximum(h, 0.0)
    h = lax.dot_general(h, w3, _CONTRACT_MINOR,
                        preferred_element_type=jnp.float32) + b3
    return jnp.maximum(h, 0.0)


def _edge_body(ef_ref, rf_ref, as_ref, ad_ref,
               w1_ref, b1_ref, w3_ref, b3_ref, w2_ref, b2_ref, out_ref):
    w1, b1 = w1_ref[...], b1_ref[...]
    w3, b3 = w3_ref[...], b3_ref[...]
    w2, b2 = w2_ref[...], b2_ref[...]
    a_s = as_ref[0]
    a_d = ad_ref[0]

    def branch(x_ref, k):
        h = _mlp2(x_ref[...], w1, b1, w3, b3)
        cf_t = lax.dot_general(w2, h, _CONTRACT_MINOR,
                               preferred_element_type=jnp.float32) + b2
        sel = lax.broadcasted_iota(jnp.int32, cf_t.shape, 0) == k
        return jnp.where(sel, cf_t, 0.0)

    part = jnp.sum(branch(ef_ref, a_s * A_ACT + a_d)
                   + branch(rf_ref, a_d * A_ACT + a_s))

    @pl.when(pl.program_id(0) == 0)
    def _():
        out_ref[...] = jnp.zeros_like(out_ref)

    out_ref[...] += part


def _node_body(nf_ref, a_ref, w1_ref, b1_ref, w3_ref, b3_ref, w2_ref, b2_ref,
               out_ref):
    h = _mlp2(nf_ref[...], w1_ref[...], b1_ref[...], w3_ref[...], b3_ref[...])
    iu_t = lax.dot_general(w2_ref[...], h, _CONTRACT_MINOR,
                           preferred_element_type=jnp.float32) + b2_ref[...]
    sel = lax.broadcasted_iota(jnp.int32, iu_t.shape, 0) == a_ref[0]
    part = jnp.sum(jnp.where(sel, iu_t, 0.0))

    @pl.when(pl.program_id(0) == 0)
    def _():
        out_ref[...] = jnp.zeros_like(out_ref)

    out_ref[...] += part


def _const_spec(shape):
    return pl.BlockSpec(shape, lambda i: (0,) * len(shape))


_edge_call = pl.pallas_call(
    _edge_body,
    grid=(_EG,),
    in_specs=[
        pl.BlockSpec((_EB, 96), lambda i: (i, 0)),
        pl.BlockSpec((_EB, 96), lambda i: (i, 0)),
        pl.BlockSpec((1, 1, _EB), lambda i: (i, 0, 0)),
        pl.BlockSpec((1, 1, _EB), lambda i: (i, 0, 0)),
        _const_spec((64, 96)),
        _const_spec((1, 64)),
        _const_spec((64, 64)),
        _const_spec((1, 64)),
        _const_spec((64, 64)),
        _const_spec((64, 1)),
    ],
    out_specs=pl.BlockSpec((1, 1), lambda i: (0, 0)),
    out_shape=jax.ShapeDtypeStruct((1, 1), jnp.float32),
    compiler_params=pltpu.CompilerParams(
        dimension_semantics=("arbitrary",)),
)

_node_call = pl.pallas_call(
    _node_body,
    grid=(_NG,),
    in_specs=[
        pl.BlockSpec((_NB, 64), lambda i: (i, 0)),
        pl.BlockSpec((1, 1, _NB), lambda i: (i, 0, 0)),
        _const_spec((64, 64)),
        _const_spec((1, 64)),
        _const_spec((64, 64)),
        _const_spec((1, 64)),
        _const_spec((A_ACT, 64)),
        _const_spec((A_ACT, 1)),
    ],
    out_specs=pl.BlockSpec((1, 1), lambda i: (0, 0)),
    out_shape=jax.ShapeDtypeStruct((1, 1), jnp.float32),
    compiler_params=pltpu.CompilerParams(
        dimension_semantics=("arbitrary",)),
)


def kernel(edge_feats_u, node_feats_u, edge_feat_reflected_u,
           ju1_w, ju1_b, ju3_w, ju3_b, ju2_w, ju2_b,
           iu1_w, iu1_b, iu3_w, iu3_b, iu2_w, iu2_b,
           edge_index, joint_acts):
    pad = _E_PAD - E_EDGES
    src = jnp.concatenate([edge_index[0], jnp.zeros((pad,), jnp.int32)])
    dst = jnp.concatenate([edge_index[1], jnp.zeros((pad,), jnp.int32)])

    a_s, a_d = _sc_gather(src, dst, joint_acts)
    a_s = a_s[:E_EDGES].reshape(_EG, 1, _EB)
    a_d = a_d[:E_EDGES].reshape(_EG, 1, _EB)

    edge_s = _edge_call(
        edge_feats_u, edge_feat_reflected_u, a_s, a_d,
        ju1_w, ju1_b.reshape(1, 64), ju3_w, ju3_b.reshape(1, 64),
        ju2_w, ju2_b.reshape(64, 1))

    node_s = _node_call(
        node_feats_u, joint_acts.reshape(_NG, 1, _NB),
        iu1_w, iu1_b.reshape(1, 64), iu3_w, iu3_b.reshape(1, 64),
        iu2_w, iu2_b.reshape(A_ACT, 1))

    return node_s + 0.25 * edge_s


# speedup vs baseline: 1.0094x; 1.0094x over previous
import jax
import jax.numpy as jnp
from jax import lax
from jax.experimental import pallas as pl
from jax.experimental.pallas import tpu as pltpu
from jax.experimental.pallas import tpu_sc as plsc

N_NODES = 50000
E_EDGES = 800000
A_ACT = 8

_NC = 2
_NS = 16
_NW = _NC * _NS
_CHUNK = 12512
_PER_W = 2 * _CHUNK
_E_PAD = _NW * _PER_W


def _sc_gather_body(src_hbm, dst_hbm, acts_hbm, as_hbm, ad_hbm,
                    src_v, dst_v, as_v, ad_v, sem):
    c = lax.axis_index("c")
    s = lax.axis_index("s")
    wid = s * _NC + c
    base = pl.multiple_of(wid * _PER_W, 8)
    pltpu.sync_copy(src_hbm.at[pl.ds(base, _PER_W)], src_v)
    pltpu.sync_copy(dst_hbm.at[pl.ds(base, _PER_W)], dst_v)
    pltpu.async_copy(acts_hbm.at[src_v], as_v, sem).wait()
    pltpu.async_copy(acts_hbm.at[dst_v], ad_v, sem).wait()
    pltpu.sync_copy(as_v, as_hbm.at[pl.ds(base, _PER_W)])
    pltpu.sync_copy(ad_v, ad_hbm.at[pl.ds(base, _PER_W)])


_sc_gather_built = None


def _sc_gather(src, dst, acts):
    global _sc_gather_built
    if _sc_gather_built is None:
        _sc_gather_built = pl.kernel(
            _sc_gather_body,
            out_type=[jax.ShapeDtypeStruct((_E_PAD,), jnp.int32),
                      jax.ShapeDtypeStruct((_E_PAD,), jnp.int32)],
            mesh=plsc.VectorSubcoreMesh(core_axis_name="c",
                                        subcore_axis_name="s"),
            scratch_types=[
                pltpu.VMEM((_PER_W,), jnp.int32),
                pltpu.VMEM((_PER_W,), jnp.int32),
                pltpu.VMEM((_PER_W,), jnp.int32),
                pltpu.VMEM((_PER_W,), jnp.int32),
                pltpu.SemaphoreType.DMA,
            ],
        )
    return _sc_gather_built(src, dst, acts)

_EB = 16000
_EG = E_EDGES // _EB
_NB = 5000
_NG = N_NODES // _NB

_CONTRACT_MINOR = (((1,), (1,)), ((), ()))


def _mlp2(x, w1, b1, w3, b3):
    h = lax.dot_general(x, w1, _CONTRACT_MINOR,
                        preferred_element_type=jnp.float32) + b1
    h = jnp.maximum(h, 0.0)
    h = lax.dot_general(h, w3, _CONTRACT_MINOR,
                        preferred_element_type=jnp.float32) + b3
    return jnp.maximum(h, 0.0)


def _edge_body(ef_ref, rf_ref, as_ref, ad_ref,
               w1_ref, b1_ref, w3_ref, b3_ref, w2_ref, b2_ref, out_ref):
    w1, b1 = w1_ref[...], b1_ref[...]
    w3, b3 = w3_ref[...], b3_ref[...]
    w2, b2 = w2_ref[...], b2_ref[...]
    off = pl.multiple_of(pl.program_id(0) * _EB, 128)
    a_s = as_ref[pl.ds(off, _EB)].reshape(1, _EB)
    a_d = ad_ref[pl.ds(off, _EB)].reshape(1, _EB)

    def branch(x_ref, k):
        h = _mlp2(x_ref[...], w1, b1, w3, b3)
        cf_t = lax.dot_general(w2, h, _CONTRACT_MINOR,
                               preferred_element_type=jnp.float32) + b2
        sel = lax.broadcasted_iota(jnp.int32, cf_t.shape, 0) == k
        return jnp.where(sel, cf_t, 0.0)

    part = jnp.sum(branch(ef_ref, a_s * A_ACT + a_d)
                   + branch(rf_ref, a_d * A_ACT + a_s))

    @pl.when(pl.program_id(0) == 0)
    def _():
        out_ref[...] = jnp.zeros_like(out_ref)

    out_ref[...] += part


def _node_body(nf_ref, a_ref, w1_ref, b1_ref, w3_ref, b3_ref, w2_ref, b2_ref,
               out_ref):
    h = _mlp2(nf_ref[...], w1_ref[...], b1_ref[...], w3_ref[...], b3_ref[...])
    iu_t = lax.dot_general(w2_ref[...], h, _CONTRACT_MINOR,
                           preferred_element_type=jnp.float32) + b2_ref[...]
    sel = lax.broadcasted_iota(jnp.int32, iu_t.shape, 0) == a_ref[0]
    part = jnp.sum(jnp.where(sel, iu_t, 0.0))

    @pl.when(pl.program_id(0) == 0)
    def _():
        out_ref[...] = jnp.zeros_like(out_ref)

    out_ref[...] += part


def _const_spec(shape):
    return pl.BlockSpec(shape, lambda i: (0,) * len(shape))


_edge_call = pl.pallas_call(
    _edge_body,
    grid=(_EG,),
    in_specs=[
        pl.BlockSpec((_EB, 96), lambda i: (i, 0)),
        pl.BlockSpec((_EB, 96), lambda i: (i, 0)),
        pl.BlockSpec((_E_PAD,), lambda i: (0,)),
        pl.BlockSpec((_E_PAD,), lambda i: (0,)),
        _const_spec((64, 96)),
        _const_spec((1, 64)),
        _const_spec((64, 64)),
        _const_spec((1, 64)),
        _const_spec((64, 64)),
        _const_spec((64, 1)),
    ],
    out_specs=pl.BlockSpec((1, 1), lambda i: (0, 0)),
    out_shape=jax.ShapeDtypeStruct((1, 1), jnp.float32),
    compiler_params=pltpu.CompilerParams(
        dimension_semantics=("arbitrary",)),
)

_node_call = pl.pallas_call(
    _node_body,
    grid=(_NG,),
    in_specs=[
        pl.BlockSpec((_NB, 64), lambda i: (i, 0)),
        pl.BlockSpec((1, 1, _NB), lambda i: (i, 0, 0)),
        _const_spec((64, 64)),
        _const_spec((1, 64)),
        _const_spec((64, 64)),
        _const_spec((1, 64)),
        _const_spec((A_ACT, 64)),
        _const_spec((A_ACT, 1)),
    ],
    out_specs=pl.BlockSpec((1, 1), lambda i: (0, 0)),
    out_shape=jax.ShapeDtypeStruct((1, 1), jnp.float32),
    compiler_params=pltpu.CompilerParams(
        dimension_semantics=("arbitrary",)),
)


def kernel(edge_feats_u, node_feats_u, edge_feat_reflected_u,
           ju1_w, ju1_b, ju3_w, ju3_b, ju2_w, ju2_b,
           iu1_w, iu1_b, iu3_w, iu3_b, iu2_w, iu2_b,
           edge_index, joint_acts):
    pad = _E_PAD - E_EDGES
    src = jnp.concatenate([edge_index[0], jnp.zeros((pad,), jnp.int32)])
    dst = jnp.concatenate([edge_index[1], jnp.zeros((pad,), jnp.int32)])

    a_s, a_d = _sc_gather(src, dst, joint_acts)

    edge_s = _edge_call(
        edge_feats_u, edge_feat_reflected_u, a_s, a_d,
        ju1_w, ju1_b.reshape(1, 64), ju3_w, ju3_b.reshape(1, 64),
        ju2_w, ju2_b.reshape(64, 1))

    node_s = _node_call(
        node_feats_u, joint_acts.reshape(_NG, 1, _NB),
        iu1_w, iu1_b.reshape(1, 64), iu3_w, iu3_b.reshape(1, 64),
        iu2_w, iu2_b.reshape(A_ACT, 1))

    return node_s + 0.25 * edge_s


# speedup vs baseline: 3.0954x; 3.0667x over previous
import jax
import jax.numpy as jnp
from jax import lax
from jax.experimental import pallas as pl
from jax.experimental.pallas import tpu as pltpu
from jax.experimental.pallas import tpu_sc as plsc

N_NODES = 50000
E_EDGES = 800000
A_ACT = 8

_NC = 2
_NS = 16
_NW = _NC * _NS
_CHUNK = 12512
_PER_W = 2 * _CHUNK
_E_PAD = _NW * _PER_W


def _sc_gather_body(src_hbm, dst_hbm, acts_hbm, as_hbm, ad_hbm,
                    src_v, dst_v, as_v, ad_v, sem):
    c = lax.axis_index("c")
    s = lax.axis_index("s")
    wid = s * _NC + c
    base = pl.multiple_of(wid * _PER_W, 8)
    pltpu.sync_copy(src_hbm.at[pl.ds(base, _PER_W)], src_v)
    pltpu.sync_copy(dst_hbm.at[pl.ds(base, _PER_W)], dst_v)
    pltpu.async_copy(acts_hbm.at[src_v], as_v, sem).wait()
    pltpu.async_copy(acts_hbm.at[dst_v], ad_v, sem).wait()
    pltpu.sync_copy(as_v, as_hbm.at[pl.ds(base, _PER_W)])
    pltpu.sync_copy(ad_v, ad_hbm.at[pl.ds(base, _PER_W)])


_sc_gather_built = None


def _sc_gather(src, dst, acts):
    global _sc_gather_built
    if _sc_gather_built is None:
        _sc_gather_built = pl.kernel(
            _sc_gather_body,
            out_type=[jax.ShapeDtypeStruct((_E_PAD,), jnp.int32),
                      jax.ShapeDtypeStruct((_E_PAD,), jnp.int32)],
            mesh=plsc.VectorSubcoreMesh(core_axis_name="c",
                                        subcore_axis_name="s"),
            scratch_types=[
                pltpu.VMEM((_PER_W,), jnp.int32),
                pltpu.VMEM((_PER_W,), jnp.int32),
                pltpu.VMEM((_PER_W,), jnp.int32),
                pltpu.VMEM((_PER_W,), jnp.int32),
                pltpu.SemaphoreType.DMA,
            ],
        )
    return _sc_gather_built(src, dst, acts)

_EB = 16000
_EG = E_EDGES // _EB

_MATMUL = (((1,), (0,)), ((), ()))


def _mlp2_t(x_t, w1, b1, w3, b3):
    h = lax.dot_general(w1, x_t, _MATMUL,
                        preferred_element_type=jnp.float32) + b1
    h = jnp.maximum(h, 0.0)
    h = lax.dot_general(w3, h, _MATMUL,
                        preferred_element_type=jnp.float32) + b3
    return jnp.maximum(h, 0.0)


def _edge_body(ef_ref, rf_ref, as_ref, ad_ref,
               w1_ref, b1_ref, w3_ref, b3_ref, w2_ref, b2_ref, out_ref):
    w1, b1 = w1_ref[...], b1_ref[...]
    w3, b3 = w3_ref[...], b3_ref[...]
    w2, b2 = w2_ref[...], b2_ref[...]
    off = pl.multiple_of(pl.program_id(0) * _EB, 128)
    a_s = as_ref[pl.ds(off, _EB)].reshape(1, _EB)
    a_d = ad_ref[pl.ds(off, _EB)].reshape(1, _EB)

    def branch(x_ref, k):
        h = _mlp2_t(x_ref[...], w1, b1, w3, b3)
        cf_t = lax.dot_general(w2, h, _MATMUL,
                               preferred_element_type=jnp.float32) + b2
        sel = lax.broadcasted_iota(jnp.int32, cf_t.shape, 0) == k
        return jnp.where(sel, cf_t, 0.0)

    part = jnp.sum(branch(ef_ref, a_s * A_ACT + a_d)
                   + branch(rf_ref, a_d * A_ACT + a_s))

    @pl.when(pl.program_id(0) == 0)
    def _():
        out_ref[...] = jnp.zeros_like(out_ref)

    out_ref[...] += part


def _node_body(nf_ref, a_ref, w1_ref, b1_ref, w3_ref, b3_ref, w2_ref, b2_ref,
               out_ref):
    h = _mlp2_t(nf_ref[...], w1_ref[...], b1_ref[...], w3_ref[...],
                b3_ref[...])
    iu_t = lax.dot_general(w2_ref[...], h, _MATMUL,
                           preferred_element_type=jnp.float32) + b2_ref[...]
    a = a_ref[...].reshape(1, N_NODES)
    sel = lax.broadcasted_iota(jnp.int32, iu_t.shape, 0) == a
    out_ref[...] = jnp.zeros_like(out_ref) + jnp.sum(jnp.where(sel, iu_t, 0.0))


def _const_spec(shape):
    return pl.BlockSpec(shape, lambda i: (0,) * len(shape))


_edge_call = pl.pallas_call(
    _edge_body,
    grid=(_EG,),
    in_specs=[
        pl.BlockSpec((96, _EB), lambda i: (0, i)),
        pl.BlockSpec((96, _EB), lambda i: (0, i)),
        pl.BlockSpec((_E_PAD,), lambda i: (0,)),
        pl.BlockSpec((_E_PAD,), lambda i: (0,)),
        _const_spec((64, 96)),
        _const_spec((64, 1)),
        _const_spec((64, 64)),
        _const_spec((64, 1)),
        _const_spec((64, 64)),
        _const_spec((64, 1)),
    ],
    out_specs=pl.BlockSpec((1, 1), lambda i: (0, 0)),
    out_shape=jax.ShapeDtypeStruct((1, 1), jnp.float32),
    compiler_params=pltpu.CompilerParams(
        dimension_semantics=("arbitrary",)),
)

_node_call = pl.pallas_call(
    _node_body,
    grid=(1,),
    in_specs=[
        pl.BlockSpec((64, N_NODES), lambda i: (0, 0)),
        pl.BlockSpec((N_NODES,), lambda i: (0,)),
        _const_spec((64, 64)),
        _const_spec((64, 1)),
        _const_spec((64, 64)),
        _const_spec((64, 1)),
        _const_spec((A_ACT, 64)),
        _const_spec((A_ACT, 1)),
    ],
    out_specs=pl.BlockSpec((1, 1), lambda i: (0, 0)),
    out_shape=jax.ShapeDtypeStruct((1, 1), jnp.float32),
    compiler_params=pltpu.CompilerParams(
        dimension_semantics=("arbitrary",)),
)


def kernel(edge_feats_u, node_feats_u, edge_feat_reflected_u,
           ju1_w, ju1_b, ju3_w, ju3_b, ju2_w, ju2_b,
           iu1_w, iu1_b, iu3_w, iu3_b, iu2_w, iu2_b,
           edge_index, joint_acts):
    pad = _E_PAD - E_EDGES
    src = jnp.concatenate([edge_index[0], jnp.zeros((pad,), jnp.int32)])
    dst = jnp.concatenate([edge_index[1], jnp.zeros((pad,), jnp.int32)])

    a_s, a_d = _sc_gather(src, dst, joint_acts)

    edge_s = _edge_call(
        edge_feats_u.T, edge_feat_reflected_u.T, a_s, a_d,
        ju1_w, ju1_b.reshape(64, 1), ju3_w, ju3_b.reshape(64, 1),
        ju2_w, ju2_b.reshape(64, 1))

    node_s = _node_call(
        node_feats_u.T, joint_acts,
        iu1_w, iu1_b.reshape(64, 1), iu3_w, iu3_b.reshape(64, 1),
        iu2_w, iu2_b.reshape(A_ACT, 1))

    return node_s + 0.25 * edge_s


# speedup vs baseline: 3.1921x; 1.0312x over previous
import jax
import jax.numpy as jnp
from jax import lax
from jax.experimental import pallas as pl
from jax.experimental.pallas import tpu as pltpu
from jax.experimental.pallas import tpu_sc as plsc

N_NODES = 50000
E_EDGES = 800000
A_ACT = 8

_NC = 2
_NS = 16
_NW = _NC * _NS
_NCHUNK = 2
_ECHUNK = E_EDGES // _NCHUNK
_PER_W = 12504
_EC_PAD = _NW * _PER_W


def _sc_gather_body(src_hbm, dst_hbm, acts_hbm, as_hbm, ad_hbm,
                    src_v, dst_v, as_v, ad_v, sem):
    c = lax.axis_index("c")
    s = lax.axis_index("s")
    wid = s * _NC + c
    base = pl.multiple_of(wid * _PER_W, 8)
    pltpu.sync_copy(src_hbm.at[pl.ds(base, _PER_W)], src_v)
    pltpu.sync_copy(dst_hbm.at[pl.ds(base, _PER_W)], dst_v)
    pltpu.async_copy(acts_hbm.at[src_v], as_v, sem).wait()
    pltpu.async_copy(acts_hbm.at[dst_v], ad_v, sem).wait()
    pltpu.sync_copy(as_v, as_hbm.at[pl.ds(base, _PER_W)])
    pltpu.sync_copy(ad_v, ad_hbm.at[pl.ds(base, _PER_W)])


_sc_gather_built = None


def _sc_gather(src, dst, acts):
    global _sc_gather_built
    if _sc_gather_built is None:
        _sc_gather_built = pl.kernel(
            _sc_gather_body,
            out_type=[jax.ShapeDtypeStruct((_EC_PAD,), jnp.int32),
                      jax.ShapeDtypeStruct((_EC_PAD,), jnp.int32)],
            mesh=plsc.VectorSubcoreMesh(core_axis_name="c",
                                        subcore_axis_name="s"),
            scratch_types=[
                pltpu.VMEM((_PER_W,), jnp.int32),
                pltpu.VMEM((_PER_W,), jnp.int32),
                pltpu.VMEM((_PER_W,), jnp.int32),
                pltpu.VMEM((_PER_W,), jnp.int32),
                pltpu.SemaphoreType.DMA,
            ],
        )
    return _sc_gather_built(src, dst, acts)

_EB = 16000
_EG = E_EDGES // _EB

_MATMUL = (((1,), (0,)), ((), ()))


def _mlp2_t(x_t, w1, b1, w3, b3):
    h = lax.dot_general(w1, x_t, _MATMUL,
                        preferred_element_type=jnp.float32) + b1
    h = jnp.maximum(h, 0.0)
    h = lax.dot_general(w3, h, _MATMUL,
                        preferred_element_type=jnp.float32) + b3
    return jnp.maximum(h, 0.0)


def _edge_body(ef_ref, rf_ref, as_ref, ad_ref,
               w1_ref, b1_ref, w3_ref, b3_ref, w2_ref, b2_ref, out_ref):
    w1, b1 = w1_ref[...], b1_ref[...]
    w3, b3 = w3_ref[...], b3_ref[...]
    w2, b2 = w2_ref[...], b2_ref[...]
    off = pl.multiple_of(pl.program_id(0) * _EB, 128)
    a_s = as_ref[pl.ds(off, _EB)].reshape(1, _EB)
    a_d = ad_ref[pl.ds(off, _EB)].reshape(1, _EB)

    def branch(x_ref, k):
        h = _mlp2_t(x_ref[...], w1, b1, w3, b3)
        cf_t = lax.dot_general(w2, h, _MATMUL,
                               preferred_element_type=jnp.float32) + b2
        sel = lax.broadcasted_iota(jnp.int32, cf_t.shape, 0) == k
        return jnp.where(sel, cf_t, 0.0)

    part = jnp.sum(branch(ef_ref, a_s * A_ACT + a_d)
                   + branch(rf_ref, a_d * A_ACT + a_s))

    @pl.when(pl.program_id(0) == 0)
    def _():
        out_ref[...] = jnp.zeros_like(out_ref)

    out_ref[...] += part


def _node_body(nf_ref, a_ref, w1_ref, b1_ref, w3_ref, b3_ref, w2_ref, b2_ref,
               out_ref):
    h = _mlp2_t(nf_ref[...], w1_ref[...], b1_ref[...], w3_ref[...],
                b3_ref[...])
    iu_t = lax.dot_general(w2_ref[...], h, _MATMUL,
                           preferred_element_type=jnp.float32) + b2_ref[...]
    a = a_ref[...].reshape(1, N_NODES)
    sel = lax.broadcasted_iota(jnp.int32, iu_t.shape, 0) == a
    out_ref[...] = jnp.zeros_like(out_ref) + jnp.sum(jnp.where(sel, iu_t, 0.0))


def _const_spec(shape):
    return pl.BlockSpec(shape, lambda i: (0,) * len(shape))


def _make_edge_call(chunk):
    blk_off = chunk * (_ECHUNK // _EB)
    return pl.pallas_call(
        _edge_body,
        grid=(_ECHUNK // _EB,),
        in_specs=[
            pl.BlockSpec((96, _EB), lambda i: (0, i + blk_off)),
            pl.BlockSpec((96, _EB), lambda i: (0, i + blk_off)),
            pl.BlockSpec((_EC_PAD,), lambda i: (0,)),
            pl.BlockSpec((_EC_PAD,), lambda i: (0,)),
            _const_spec((64, 96)),
            _const_spec((64, 1)),
            _const_spec((64, 64)),
            _const_spec((64, 1)),
            _const_spec((64, 64)),
            _const_spec((64, 1)),
        ],
        out_specs=pl.BlockSpec((1, 1), lambda i: (0, 0)),
        out_shape=jax.ShapeDtypeStruct((1, 1), jnp.float32),
        compiler_params=pltpu.CompilerParams(
            dimension_semantics=("arbitrary",)),
    )


_edge_calls = [_make_edge_call(c) for c in range(_NCHUNK)]

_node_call = pl.pallas_call(
    _node_body,
    grid=(1,),
    in_specs=[
        pl.BlockSpec((64, N_NODES), lambda i: (0, 0)),
        pl.BlockSpec((N_NODES,), lambda i: (0,)),
        _const_spec((64, 64)),
        _const_spec((64, 1)),
        _const_spec((64, 64)),
        _const_spec((64, 1)),
        _const_spec((A_ACT, 64)),
        _const_spec((A_ACT, 1)),
    ],
    out_specs=pl.BlockSpec((1, 1), lambda i: (0, 0)),
    out_shape=jax.ShapeDtypeStruct((1, 1), jnp.float32),
    compiler_params=pltpu.CompilerParams(
        dimension_semantics=("arbitrary",)),
)


def kernel(edge_feats_u, node_feats_u, edge_feat_reflected_u,
           ju1_w, ju1_b, ju3_w, ju3_b, ju2_w, ju2_b,
           iu1_w, iu1_b, iu3_w, iu3_b, iu2_w, iu2_b,
           edge_index, joint_acts):
    pad = jnp.zeros((_EC_PAD - _ECHUNK,), jnp.int32)
    edge_s = None
    for c in range(_NCHUNK):
        sl = slice(c * _ECHUNK, (c + 1) * _ECHUNK)
        a_s, a_d = _sc_gather(jnp.concatenate([edge_index[0, sl], pad]),
                              jnp.concatenate([edge_index[1, sl], pad]),
                              joint_acts)
        part = _edge_calls[c](
            edge_feats_u.T, edge_feat_reflected_u.T, a_s, a_d,
            ju1_w, ju1_b.reshape(64, 1), ju3_w, ju3_b.reshape(64, 1),
            ju2_w, ju2_b.reshape(64, 1))
        edge_s = part if edge_s is None else edge_s + part

    node_s = _node_call(
        node_feats_u.T, joint_acts,
        iu1_w, iu1_b.reshape(64, 1), iu3_w, iu3_b.reshape(64, 1),
        iu2_w, iu2_b.reshape(A_ACT, 1))

    return node_s + 0.25 * edge_s
